# baseline (speedup 1.0000x reference)
import jax
import jax.numpy as jnp
from jax.experimental import pallas as pl

N = 10000
G = 64


def _gat(x, src, dst, W, a_src, a_dst, b):
    n = x.shape[0]
    h = x @ W
    loop = jnp.arange(n, dtype=src.dtype)
    src = jnp.concatenate([src, loop])
    dst = jnp.concatenate([dst, loop])
    e = (h @ a_src)[src] + (h @ a_dst)[dst]
    e = jax.nn.leaky_relu(e, 0.2)
    m = jax.ops.segment_max(e, dst, num_segments=n)
    e = jnp.exp(e - m[dst])
    denom = jax.ops.segment_sum(e, dst, num_segments=n)
    alpha = e / (denom[dst] + 1e-16)
    out = jax.ops.segment_sum(h[src] * alpha[:, None], dst, num_segments=n)
    return out + b


def _head_body(flat_ref, wo_ref, bo_ref, out_ref):
    o = flat_ref[...] @ wo_ref[...] + bo_ref[...]
    m = jnp.max(o, axis=-1, keepdims=True)
    s = jnp.log(jnp.sum(jnp.exp(o - m), axis=-1, keepdims=True))
    out_ref[...] = o - m - s


def kernel(x, edges_idx, batch_idx, g_features, W1, a_src1, a_dst1, b1, W2, a_src2, a_dst2, b2, Wg, bg, Wo, bo):
    src = edges_idx[0]
    dst = edges_idx[1]
    h = _gat(x, src, dst, W1, a_src1, a_dst1, b1)
    h = _gat(h, src, dst, W2, a_src2, a_dst2, b2)
    ones = jnp.ones((h.shape[0],), dtype=h.dtype)
    cnt = jax.ops.segment_sum(ones, batch_idx, num_segments=G)
    mean_p = jax.ops.segment_sum(h, batch_idx, num_segments=G) / jnp.maximum(cnt, 1.0)[:, None]
    max_p = jax.ops.segment_max(h, batch_idx, num_segments=G)
    g_ft = g_features @ Wg + bg
    flat = jnp.concatenate([mean_p, max_p, g_ft], axis=1)
    out = pl.pallas_call(
        _head_body,
        out_shape=jax.ShapeDtypeStruct((G, 2), jnp.float32),
    )(flat, Wo, bo[None, :])
    return out


# speedup vs baseline: 17.2357x; 17.2357x over previous
import functools

import jax
import jax.numpy as jnp
from jax import lax
from jax.experimental import pallas as pl
from jax.experimental.pallas import tpu as pltpu
from jax.experimental.pallas import tpu_sc as plsc

N = 10000
NP = 10240
E = 320000
D = 128
G = 64
NC = 2
NS = 16
NW = NC * NS
EPW = E // NW
CW = 128
CH = (EPW + CW - 1) // CW
EPW_PAD = 10240
CHP = EPW_PAD // CW
RPS = NP // NS

_f32 = jnp.float32
_i32 = jnp.int32


def _attn_prep(h, asv, adv):
    a_s = h @ asv
    a_d = h @ adv
    c = jnp.max(a_s) + jnp.max(a_d)
    m = jnp.maximum(c, 0.2 * c)
    es = a_s + a_d
    p_self = jnp.exp(jnp.maximum(es, 0.2 * es) - m)
    return a_s, a_d, p_self, m


def _write_ext(ref, val, pad_val):
    ref[0:N, :] = val
    ref[N:NP, :] = jnp.full((NP - N, val.shape[1]), pad_val, val.dtype)


def _prep_body(x_ref, w_ref, asv_ref, adv_ref,
               h_out, as_out, ad_out, ps_out, m_out):
    h = x_ref[...] @ w_ref[...]
    a_s, a_d, p_self, m = _attn_prep(h, asv_ref[...], adv_ref[...])
    _write_ext(h_out, h, 0.0)
    _write_ext(as_out, a_s, -1e30)
    _write_ext(ad_out, a_d, -1e30)
    _write_ext(ps_out, p_self, 0.0)
    m_out[...] = jnp.full((16, 1), m, _f32)


def _normalize(outp_ref, denp_ref, ps_ref, h_ref, b_ref):
    num = outp_ref[0] + outp_ref[1] + ps_ref[...] * h_ref[...]
    den = denp_ref[0] + denp_ref[1] + ps_ref[...]
    return num / (den + 1e-16) + b_ref[...]


def _combine_body(outp_ref, denp_ref, ps_ref, h_ref, b_ref,
                  w2_ref, asv_ref, adv_ref,
                  h2_out, as_out, ad_out, ps_out, m_out):
    h1 = _normalize(outp_ref, denp_ref, ps_ref, h_ref, b_ref)
    h2 = h1[0:N, :] @ w2_ref[...]
    a_s, a_d, p_self, m = _attn_prep(h2, asv_ref[...], adv_ref[...])
    _write_ext(h2_out, h2, 0.0)
    _write_ext(as_out, a_s, -1e30)
    _write_ext(ad_out, a_d, -1e30)
    _write_ext(ps_out, p_self, 0.0)
    m_out[...] = jnp.full((16, 1), m, _f32)


def _combine2_body(outp_ref, denp_ref, ps_ref, h_ref, b_ref, h2n_out):
    h2n_out[...] = _normalize(outp_ref, denp_ref, ps_ref, h_ref, b_ref)


def _head_body(h2n_ref, maxp_ref, batch_ref,
               gf_ref, wg_ref, bg_ref, wo_ref, bo_ref, out_ref):
    h2 = h2n_ref[...]
    batch = batch_ref[...]
    gids = lax.broadcasted_iota(_i32, (G, NP), 0)
    onehot = (jnp.broadcast_to(batch, (G, NP)) == gids).astype(_f32)
    cnt = jnp.sum(onehot, axis=1, keepdims=True)
    sum_p = jax.lax.dot(onehot, h2,
                        preferred_element_type=_f32)
    mean_p = sum_p / jnp.maximum(cnt, 1.0)
    acc = maxp_ref[0]
    for i in range(1, NW):
        acc = jnp.maximum(acc, maxp_ref[i])
    g_ft = gf_ref[...] @ wg_ref[...] + bg_ref[...]
    flat = jnp.concatenate([mean_p, acc, g_ft], axis=1)
    o = flat @ wo_ref[...] + bo_ref[...]
    mx = jnp.max(o, axis=-1, keepdims=True)
    out_ref[...] = o - mx - jnp.log(jnp.sum(jnp.exp(o - mx), axis=-1,
                                            keepdims=True))


def _sc_edge_body(h_hbm, srcg_hbm, dstg_hbm, as_hbm, ad_hbm, m_hbm,
                  z2_hbm, z1_hbm,
                  outp_hbm, denp_hbm,
                  src_v, dst_v, p_c, asg_v, adg_v, m_v, rows_v,
                  out_sh, den_sh, as_sh, ad_sh, gsem, asem):
    c = lax.axis_index("c")
    s = lax.axis_index("s")
    w = c * NS + s

    base = s * RPS
    pltpu.sync_copy(z2_hbm.at[pl.ds(base, RPS)], out_sh.at[pl.ds(base, RPS)])
    pltpu.sync_copy(z1_hbm.at[pl.ds(base, RPS)], den_sh.at[pl.ds(base, RPS)])

    @pl.when(s == 0)
    def _():
        pltpu.sync_copy(as_hbm, as_sh)
        pltpu.sync_copy(ad_hbm, ad_sh)
    pltpu.sync_copy(srcg_hbm.at[w], src_v)
    pltpu.sync_copy(dstg_hbm.at[w], dst_v)
    pltpu.sync_copy(m_hbm, m_v)
    plsc.subcore_barrier()

    m_vec = m_v[...]

    def row_chunk(j, _):
        rcp = pltpu.async_copy(h_hbm.at[src_v.at[j]], rows_v, gsem)
        acp = pltpu.async_copy(as_sh.at[src_v.at[j]], asg_v, asem)
        bcp = pltpu.async_copy(ad_sh.at[dst_v.at[j]], adg_v, asem)
        acp.wait()
        bcp.wait()
        for k in range(CW // 16):
            sl = pl.ds(k * 16, 16)
            e = asg_v[sl] + adg_v[sl]
            e = jnp.maximum(e, 0.2 * e)
            p_c[sl] = jnp.exp(e - m_vec)
        rcp.wait()

        def scale_edge(e_i, _):
            pb = plsc.load_gather(p_c, [jnp.full((16,), e_i, _i32)])
            for d8 in range(D // 16):
                sl = pl.ds(d8 * 16, 16)
                rows_v[e_i, sl] = rows_v[e_i, sl] * pb
            return 0
        lax.fori_loop(0, CW, scale_edge, 0)

        pltpu.sync_copy(rows_v, out_sh.at[dst_v.at[j]], add=True)
        pltpu.sync_copy(p_c, den_sh.at[dst_v.at[j]], add=True)
        return 0
    lax.fori_loop(0, CHP, row_chunk, 0)

    plsc.subcore_barrier()
    pltpu.sync_copy(out_sh.at[pl.ds(base, RPS)],
                    outp_hbm.at[c, pl.ds(base, RPS)])
    pltpu.sync_copy(den_sh.at[pl.ds(base, RPS)],
                    denp_hbm.at[c, pl.ds(base, RPS)])


_sc_edge = pl.kernel(
    _sc_edge_body,
    out_type=(jax.ShapeDtypeStruct((NC, NP, D), _f32),
              jax.ShapeDtypeStruct((NC, NP), _f32)),
    mesh=plsc.VectorSubcoreMesh(core_axis_name="c", subcore_axis_name="s",
                                num_cores=NC, num_subcores=NS),
    compiler_params=pltpu.CompilerParams(needs_layout_passes=False),
    scratch_types=[
        pltpu.VMEM((CHP, CW), _i32),
        pltpu.VMEM((CHP, CW), _i32),
        pltpu.VMEM((CW,), _f32),
        pltpu.VMEM((CW,), _f32),
        pltpu.VMEM((CW,), _f32),
        pltpu.VMEM((16,), _f32),
        pltpu.VMEM((CW, D), _f32),
        pltpu.VMEM_SHARED((NP, D), _f32),
        pltpu.VMEM_SHARED((NP,), _f32),
        pltpu.VMEM_SHARED((NP,), _f32),
        pltpu.VMEM_SHARED((NP,), _f32),
        pltpu.SemaphoreType.DMA,
        pltpu.SemaphoreType.DMA,
    ],
)


RPW = NP // NW


def _sc_pool_body(h_hbm, batch_hbm, maxp_hbm, rows_v, batch_v, acc_v, psem):
    c = lax.axis_index("c")
    s = lax.axis_index("s")
    w = c * NS + s

    cp = pltpu.async_copy(h_hbm.at[pl.ds(w * RPW, RPW)], rows_v, psem)
    pltpu.sync_copy(batch_hbm.at[w], batch_v)

    neg = jnp.full((16,), -jnp.inf, _f32)

    def init_row(g, _):
        for d8 in range(D // 16):
            acc_v[g, pl.ds(d8 * 16, 16)] = neg
        return 0
    lax.fori_loop(0, G + 1, init_row, 0)
    cp.wait()

    def pool_row16(t, _):
        bv = batch_v[pl.ds(t * 16, 16)]
        for k in range(16):
            g = bv[k]
            r = t * 16 + k
            for d8 in range(D // 16):
                sl = pl.ds(d8 * 16, 16)
                acc_v[g, sl] = jnp.maximum(acc_v[g, sl], rows_v[r, sl])
        return 0
    lax.fori_loop(0, RPW // 16, pool_row16, 0)

    pltpu.sync_copy(acc_v.at[pl.ds(0, G)], maxp_hbm.at[w])


_sc_pool = pl.kernel(
    _sc_pool_body,
    out_type=jax.ShapeDtypeStruct((NW, G, D), _f32),
    mesh=plsc.VectorSubcoreMesh(core_axis_name="c", subcore_axis_name="s",
                                num_cores=NC, num_subcores=NS),
    compiler_params=pltpu.CompilerParams(needs_layout_passes=False),
    scratch_types=[
        pltpu.VMEM((RPW, D), _f32),
        pltpu.VMEM((RPW,), _i32),
        pltpu.VMEM((G + 1, D), _f32),
        pltpu.SemaphoreType.DMA,
    ],
)


def _tc(body, out_shape):
    return pl.pallas_call(body, out_shape=out_shape)


def kernel(x, edges_idx, batch_idx, g_features, W1, a_src1, a_dst1, b1,
           W2, a_src2, a_dst2, b2, Wg, bg, Wo, bo):
    src = edges_idx[0].reshape(NW, EPW)
    dst = edges_idx[1].reshape(NW, EPW)
    pad_s = jnp.full((NW, EPW_PAD - EPW), N, _i32)
    pad_d = jnp.zeros((NW, EPW_PAD - EPW), _i32)
    srcg = jnp.concatenate([src, pad_s], axis=1).reshape(NW, CHP, CW)
    dstg = jnp.concatenate([dst, pad_d], axis=1).reshape(NW, CHP, CW)
    z2 = jnp.zeros((NP, D), _f32)
    z1 = jnp.zeros((NP,), _f32)

    ext = jax.ShapeDtypeStruct((NP, 1), _f32)
    prep_out = [jax.ShapeDtypeStruct((NP, D), _f32), ext, ext, ext,
                jax.ShapeDtypeStruct((16, 1), _f32)]

    h1e, as1e, ad1e, ps1, m1 = _tc(_prep_body, prep_out)(
        x, W1, a_src1[:, None], a_dst1[:, None])

    outp1, denp1 = _sc_edge(h1e, srcg, dstg, as1e.reshape(NP),
                            ad1e.reshape(NP), m1.reshape(16), z2, z1)

    h2e, as2e, ad2e, ps2, m2 = _tc(_combine_body, prep_out)(
        outp1, denp1[:, :, None], ps1, h1e, b1[None, :],
        W2, a_src2[:, None], a_dst2[:, None])

    outp2, denp2 = _sc_edge(h2e, srcg, dstg, as2e.reshape(NP),
                            ad2e.reshape(NP), m2.reshape(16), z2, z1)

    h2n = _tc(_combine2_body, jax.ShapeDtypeStruct((NP, D), _f32))(
        outp2, denp2[:, :, None], ps2, h2e, b2[None, :])

    batch_pad = jnp.concatenate(
        [batch_idx, jnp.full((NP - N,), G, _i32)])

    maxp = _sc_pool(h2n, batch_pad.reshape(NW, RPW))

    out = _tc(_head_body, jax.ShapeDtypeStruct((G, 2), _f32))(
        h2n, maxp, batch_pad[None, :],
        g_features, Wg, bg[None, :], Wo, bo[None, :])
    return out


# speedup vs baseline: 17.4548x; 1.0127x over previous
import functools

import jax
import jax.numpy as jnp
from jax import lax
from jax.experimental import pallas as pl
from jax.experimental.pallas import tpu as pltpu
from jax.experimental.pallas import tpu_sc as plsc

N = 10000
NP = 10240
E = 320000
D = 128
G = 64
NC = 2
NS = 16
NW = NC * NS
EPW = E // NW
CW = 128
NCH = 80
EPW_PAD = NCH * CW
RPS = NP // NS

_f32 = jnp.float32
_i32 = jnp.int32


def _attn_prep(h, asv, adv):
    a_s = h @ asv
    a_d = h @ adv
    c = jnp.max(a_s) + jnp.max(a_d)
    m = jnp.maximum(c, 0.2 * c)
    es = a_s + a_d
    p_self = jnp.exp(jnp.maximum(es, 0.2 * es) - m)
    return a_s, a_d, p_self, m


def _write_ext(ref, val, pad_val):
    ref[0:N, :] = val
    ref[N:NP, :] = jnp.full((NP - N, val.shape[1]), pad_val, val.dtype)


def _prep_body(x_ref, w_ref, asv_ref, adv_ref,
               h_out, as_out, ad_out, ps_out, m_out):
    h = x_ref[...] @ w_ref[...]
    a_s, a_d, p_self, m = _attn_prep(h, asv_ref[...], adv_ref[...])
    _write_ext(h_out, h, 0.0)
    _write_ext(as_out, a_s, -1e30)
    _write_ext(ad_out, a_d, -1e30)
    _write_ext(ps_out, p_self, 0.0)
    m_out[...] = jnp.full((16, 1), m, _f32)


def _normalize(outp_ref, denp_ref, ps_ref, h_ref, b_ref):
    num = outp_ref[0] + outp_ref[1] + ps_ref[...] * h_ref[...]
    den = denp_ref[0] + denp_ref[1] + ps_ref[...]
    return num / (den + 1e-16) + b_ref[...]


def _combine_body(outp_ref, denp_ref, ps_ref, h_ref, b_ref,
                  w2_ref, asv_ref, adv_ref,
                  h2_out, as_out, ad_out, ps_out, m_out):
    h1 = _normalize(outp_ref, denp_ref, ps_ref, h_ref, b_ref)
    h2 = h1[0:N, :] @ w2_ref[...]
    a_s, a_d, p_self, m = _attn_prep(h2, asv_ref[...], adv_ref[...])
    _write_ext(h2_out, h2, 0.0)
    _write_ext(as_out, a_s, -1e30)
    _write_ext(ad_out, a_d, -1e30)
    _write_ext(ps_out, p_self, 0.0)
    m_out[...] = jnp.full((16, 1), m, _f32)


def _combine2_body(outp_ref, denp_ref, ps_ref, h_ref, b_ref, h2n_out):
    h2n_out[...] = _normalize(outp_ref, denp_ref, ps_ref, h_ref, b_ref)


def _head_body(h2n_ref, maxp_ref, batch_ref,
               gf_ref, wg_ref, bg_ref, wo_ref, bo_ref, out_ref):
    h2 = h2n_ref[...]
    batch = batch_ref[...]
    gids = lax.broadcasted_iota(_i32, (G, NP), 0)
    onehot = (jnp.broadcast_to(batch, (G, NP)) == gids).astype(_f32)
    cnt = jnp.sum(onehot, axis=1, keepdims=True)
    sum_p = jax.lax.dot(onehot, h2,
                        preferred_element_type=_f32)
    mean_p = sum_p / jnp.maximum(cnt, 1.0)
    acc = maxp_ref[0]
    for i in range(1, NW):
        acc = jnp.maximum(acc, maxp_ref[i])
    g_ft = gf_ref[...] @ wg_ref[...] + bg_ref[...]
    flat = jnp.concatenate([mean_p, acc, g_ft], axis=1)
    o = flat @ wo_ref[...] + bo_ref[...]
    mx = jnp.max(o, axis=-1, keepdims=True)
    out_ref[...] = o - mx - jnp.log(jnp.sum(jnp.exp(o - mx), axis=-1,
                                            keepdims=True))


def _sc_edge_body(h_hbm, srcg_hbm, dstg_hbm, as_hbm, ad_hbm, m_hbm,
                  z2_hbm, z1_hbm,
                  outp_hbm, denp_hbm,
                  src_v, dst_v, p_c, asg_v, adg_v, m_v, rows_v,
                  out_sh, den_sh, as_sh, ad_sh, gsem, asem):
    c = lax.axis_index("c")
    s = lax.axis_index("s")
    w = c * NS + s

    base = s * RPS
    pltpu.sync_copy(z2_hbm.at[pl.ds(base, RPS)], out_sh.at[pl.ds(base, RPS)])
    pltpu.sync_copy(z1_hbm.at[pl.ds(base, RPS)], den_sh.at[pl.ds(base, RPS)])

    @pl.when(s == 0)
    def _():
        pltpu.sync_copy(as_hbm, as_sh)
        pltpu.sync_copy(ad_hbm, ad_sh)
    pltpu.sync_copy(srcg_hbm.at[w], src_v)
    pltpu.sync_copy(dstg_hbm.at[w], dst_v)
    pltpu.sync_copy(m_hbm, m_v)
    plsc.subcore_barrier()

    m_vec = m_v[...]

    def row_chunk(j, _):
        rcp = pltpu.async_copy(h_hbm.at[src_v.at[j]], rows_v, gsem)
        acp = pltpu.async_copy(as_sh.at[src_v.at[j]], asg_v, asem)
        bcp = pltpu.async_copy(ad_sh.at[dst_v.at[j]], adg_v, asem)
        acp.wait()
        bcp.wait()
        for k in range(CW // 16):
            sl = pl.ds(k * 16, 16)
            e = asg_v[sl] + adg_v[sl]
            e = jnp.maximum(e, 0.2 * e)
            p_c[sl] = jnp.exp(e - m_vec)
        rcp.wait()

        def scale2(e2, _):
            for ei in (2 * e2, 2 * e2 + 1):
                pb = plsc.load_gather(p_c, [jnp.full((16,), ei, _i32)])
                for d8 in range(D // 16):
                    sl = pl.ds(d8 * 16, 16)
                    rows_v[ei, sl] = rows_v[ei, sl] * pb
            return 0
        lax.fori_loop(0, CW // 2, scale2, 0)

        pltpu.sync_copy(rows_v, out_sh.at[dst_v.at[j]], add=True)
        pltpu.sync_copy(p_c, den_sh.at[dst_v.at[j]], add=True)
        return 0
    lax.fori_loop(0, NCH, row_chunk, 0)

    plsc.subcore_barrier()
    pltpu.sync_copy(out_sh.at[pl.ds(base, RPS)],
                    outp_hbm.at[c, pl.ds(base, RPS)])
    pltpu.sync_copy(den_sh.at[pl.ds(base, RPS)],
                    denp_hbm.at[c, pl.ds(base, RPS)])


_sc_edge = pl.kernel(
    _sc_edge_body,
    out_type=(jax.ShapeDtypeStruct((NC, NP, D), _f32),
              jax.ShapeDtypeStruct((NC, NP), _f32)),
    mesh=plsc.VectorSubcoreMesh(core_axis_name="c", subcore_axis_name="s",
                                num_cores=NC, num_subcores=NS),
    compiler_params=pltpu.CompilerParams(needs_layout_passes=False),
    scratch_types=[
        pltpu.VMEM((NCH, CW), _i32),
        pltpu.VMEM((NCH, CW), _i32),
        pltpu.VMEM((CW,), _f32),
        pltpu.VMEM((CW,), _f32),
        pltpu.VMEM((CW,), _f32),
        pltpu.VMEM((16,), _f32),
        pltpu.VMEM((CW, D), _f32),
        pltpu.VMEM_SHARED((NP, D), _f32),
        pltpu.VMEM_SHARED((NP,), _f32),
        pltpu.VMEM_SHARED((NP,), _f32),
        pltpu.VMEM_SHARED((NP,), _f32),
    ] + [pltpu.SemaphoreType.DMA] * 2,
)


RPW = NP // NW


def _sc_pool_body(h_hbm, batch_hbm, maxp_hbm, rows_v, batch_v, acc_v, psem):
    c = lax.axis_index("c")
    s = lax.axis_index("s")
    w = c * NS + s

    cp = pltpu.async_copy(h_hbm.at[pl.ds(w * RPW, RPW)], rows_v, psem)
    pltpu.sync_copy(batch_hbm.at[w], batch_v)

    neg = jnp.full((16,), -jnp.inf, _f32)

    def init_row(g, _):
        for d8 in range(D // 16):
            acc_v[g, pl.ds(d8 * 16, 16)] = neg
        return 0
    lax.fori_loop(0, G + 1, init_row, 0)
    cp.wait()

    def pool_row16(t, _):
        bv = batch_v[pl.ds(t * 16, 16)]
        for k in range(16):
            g = bv[k]
            r = t * 16 + k
            for d8 in range(D // 16):
                sl = pl.ds(d8 * 16, 16)
                acc_v[g, sl] = jnp.maximum(acc_v[g, sl], rows_v[r, sl])
        return 0
    lax.fori_loop(0, RPW // 16, pool_row16, 0)

    pltpu.sync_copy(acc_v.at[pl.ds(0, G)], maxp_hbm.at[w])


_sc_pool = pl.kernel(
    _sc_pool_body,
    out_type=jax.ShapeDtypeStruct((NW, G, D), _f32),
    mesh=plsc.VectorSubcoreMesh(core_axis_name="c", subcore_axis_name="s",
                                num_cores=NC, num_subcores=NS),
    compiler_params=pltpu.CompilerParams(needs_layout_passes=False),
    scratch_types=[
        pltpu.VMEM((RPW, D), _f32),
        pltpu.VMEM((RPW,), _i32),
        pltpu.VMEM((G + 1, D), _f32),
        pltpu.SemaphoreType.DMA,
    ],
)


def _tc(body, out_shape):
    return pl.pallas_call(body, out_shape=out_shape)


def kernel(x, edges_idx, batch_idx, g_features, W1, a_src1, a_dst1, b1,
           W2, a_src2, a_dst2, b2, Wg, bg, Wo, bo):
    src = edges_idx[0].reshape(NW, EPW)
    dst = edges_idx[1].reshape(NW, EPW)
    pad_s = jnp.full((NW, EPW_PAD - EPW), N, _i32)
    pad_d = jnp.zeros((NW, EPW_PAD - EPW), _i32)
    srcg = jnp.concatenate([src, pad_s], axis=1).reshape(NW, NCH, CW)
    dstg = jnp.concatenate([dst, pad_d], axis=1).reshape(NW, NCH, CW)
    z2 = jnp.zeros((NP, D), _f32)
    z1 = jnp.zeros((NP,), _f32)

    ext = jax.ShapeDtypeStruct((NP, 1), _f32)
    prep_out = [jax.ShapeDtypeStruct((NP, D), _f32), ext, ext, ext,
                jax.ShapeDtypeStruct((16, 1), _f32)]

    h1e, as1e, ad1e, ps1, m1 = _tc(_prep_body, prep_out)(
        x, W1, a_src1[:, None], a_dst1[:, None])

    outp1, denp1 = _sc_edge(h1e, srcg, dstg, as1e.reshape(NP),
                            ad1e.reshape(NP), m1.reshape(16), z2, z1)

    h2e, as2e, ad2e, ps2, m2 = _tc(_combine_body, prep_out)(
        outp1, denp1[:, :, None], ps1, h1e, b1[None, :],
        W2, a_src2[:, None], a_dst2[:, None])

    outp2, denp2 = _sc_edge(h2e, srcg, dstg, as2e.reshape(NP),
                            ad2e.reshape(NP), m2.reshape(16), z2, z1)

    h2n = _tc(_combine2_body, jax.ShapeDtypeStruct((NP, D), _f32))(
        outp2, denp2[:, :, None], ps2, h2e, b2[None, :])

    batch_pad = jnp.concatenate(
        [batch_idx, jnp.full((NP - N,), G, _i32)])

    maxp = _sc_pool(h2n, batch_pad.reshape(NW, RPW))

    out = _tc(_head_body, jax.ShapeDtypeStruct((G, 2), _f32))(
        h2n, maxp, batch_pad[None, :],
        g_features, Wg, bg[None, :], Wo, bo[None, :])
    return out
